# baseline (speedup 1.0000x reference)
import numpy as np
import jax
import jax.numpy as jnp
from jax import lax
from jax.experimental import pallas as pl
from jax.experimental.pallas import tpu as pltpu
from jax.experimental.pallas import tpu_sc as plsc

VOCAB = 100000
DIM = 128
BASE = 512
NC, NS = 2, 16
NW = NC * NS
LANES = 16
NB = DIM // LANES
MAGIC = 12582912.0
S0, S1 = 96, 104


def _positional_table(seq_len):
    position = np.arange(BASE, dtype=np.float32)[:, None]
    div_term = np.exp(
        np.arange(0, DIM, 2, dtype=np.float32) * (-np.log(10000.0) / DIM))
    pe = np.zeros((BASE, DIM), dtype=np.float32)
    pe[:, 0::2] = np.sin(position * div_term)
    pe[:, 1::2] = np.cos(position * div_term)
    return jnp.asarray(pe[:seq_len])


def _minmax_body(w_ref, mn_ref, mx_ref):
    i = pl.program_id(0)
    bmn = jnp.min(w_ref[...])
    bmx = jnp.max(w_ref[...])

    @pl.when(i == 0)
    def _():
        mn_ref[0, 0] = bmn
        mx_ref[0, 0] = bmx

    @pl.when(i != 0)
    def _():
        mn_ref[0, 0] = jnp.minimum(mn_ref[0, 0], bmn)
        mx_ref[0, 0] = jnp.maximum(mx_ref[0, 0], bmx)


def _weight_minmax(weight):
    rows = 2000
    grid = VOCAB // rows
    mn, mx = pl.pallas_call(
        _minmax_body,
        grid=(grid,),
        in_specs=[pl.BlockSpec((rows, DIM), lambda i: (i, 0))],
        out_specs=[
            pl.BlockSpec((1, 1), lambda i: (0, 0), memory_space=pltpu.SMEM),
            pl.BlockSpec((1, 1), lambda i: (0, 0), memory_space=pltpu.SMEM),
        ],
        out_shape=[
            jax.ShapeDtypeStruct((1, 1), jnp.float32),
            jax.ShapeDtypeStruct((1, 1), jnp.float32),
        ],
    )(weight)
    return mn[0, 0], mx[0, 0]


def _rsqrt_scalar(x):
    i = lax.bitcast_convert_type(x, jnp.int32)
    i = jnp.int32(0x5F3759DF) - (i >> 1)
    y = lax.bitcast_convert_type(i, jnp.float32)
    for _ in range(3):
        y = y * (1.5 - 0.5 * x * y * y)
    return y


def _make_sc_kernel(tokens, seq):
    tok_w = tokens // NW
    ch = seq
    chp = (ch + LANES - 1) // LANES * LANES
    nchunk = tok_w // ch
    assert tok_w % ch == 0 and tokens % NW == 0 and nchunk >= 3

    mesh = plsc.VectorSubcoreMesh(core_axis_name="c", subcore_axis_name="s")

    def body(ids_hbm, w_hbm, pe_hbm, gam_hbm, bet_hbm, cst_hbm, out_hbm,
             idx0, idx1, idx2, rows0, rows1, rows2, pev, et, gv, bv, cv,
             gsem0, gsem1, gsem2, ssem0, ssem1, ssem2):
        idxs = (idx0, idx1, idx2)
        rows = (rows0, rows1, rows2)
        gsems = (gsem0, gsem1, gsem2)
        ssems = (ssem0, ssem1, ssem2)

        cid = lax.axis_index("c")
        sid = lax.axis_index("s")
        wid = sid * NC + cid
        tok0 = wid * tok_w

        pltpu.sync_copy(pe_hbm, pev)
        pltpu.sync_copy(gam_hbm, gv)
        pltpu.sync_copy(bet_hbm, bv)
        pltpu.sync_copy(cst_hbm, cv)

        inv = cv[pl.ds(0, LANES)]
        zp = cv[pl.ds(LANES, LANES)]
        scale = cv[pl.ds(2 * LANES, LANES)]
        zps = zp * scale

        def pe_body(c, carry):
            for j in range(chp // LANES):
                sl = pl.ds(j * LANES, LANES)
                pev[c, sl] = pev[c, sl] - zps
            return carry
        lax.fori_loop(0, DIM, pe_body, 0)

        def start_gather(c, idxb, rowsb, gsem):
            t0 = tok0 + c * ch
            pltpu.sync_copy(ids_hbm.at[pl.ds(t0, ch)], idxb)
            pltpu.async_copy(w_hbm.at[idxb.at[pl.ds(0, S0)]],
                             rowsb.at[pl.ds(0, S0)], gsem)
            pltpu.async_copy(w_hbm.at[idxb.at[pl.ds(S0, S1)]],
                             rowsb.at[pl.ds(S0, S1)], gsem)

        def wait_gather(idxb, rowsb, gsem):
            pltpu.make_async_copy(w_hbm.at[idxb.at[pl.ds(0, S0)]],
                                  rowsb.at[pl.ds(0, S0)], gsem).wait()
            pltpu.make_async_copy(w_hbm.at[idxb.at[pl.ds(S0, S1)]],
                                  rowsb.at[pl.ds(S0, S1)], gsem).wait()

        def start_scatter(c, rowsb, ssem):
            t0 = tok0 + c * ch
            pltpu.async_copy(rowsb.at[pl.ds(0, ch)], out_hbm.at[pl.ds(t0, ch)],
                             ssem)

        def wait_scatter(c, rowsb, ssem):
            t0 = tok0 + c * ch
            pltpu.make_async_copy(rowsb.at[pl.ds(0, ch)],
                                  out_hbm.at[pl.ds(t0, ch)], ssem).wait()

        CU = 8
        NACC = 4
        iota = lax.iota(jnp.int32, LANES)
        zero = jnp.zeros((LANES,), jnp.float32)

        def compute_chunk(rowsb, et):
            def group_body(gidx, carry):
                r0 = gidx * LANES
                rowv = iota + r0
                def col_body(i, acc):
                    accs = list(acc)
                    for u in range(CU):
                        col = i * CU + u
                        colv = jnp.full((LANES,), col, jnp.int32)
                        x = plsc.load_gather(rowsb, [rowv, colv])
                        y = x * inv + zp
                        q = (y + MAGIC) - MAGIC
                        e = q * scale + pev[col, pl.ds(r0, LANES)]
                        et[col, :] = e
                        a = u % NACC
                        vs, vq = accs[a]
                        accs[a] = (vs + e, vq + e * e)
                    return tuple(accs)
                accs = lax.fori_loop(
                    0, DIM // CU, col_body, tuple((zero, zero) for _ in range(NACC)))
                vsum = accs[0][0]
                vsq = accs[0][1]
                for a in range(1, NACC):
                    vsum = vsum + accs[a][0]
                    vsq = vsq + accs[a][1]
                mean = vsum * (1.0 / DIM)
                var = vsq * (1.0 / DIM) - mean * mean
                xin = var + 1e-5
                bi = plsc.bitcast(xin, jnp.int32)
                bi = jnp.full((LANES,), 0x5F3759DF, jnp.int32) - (bi >> 1)
                rstd = plsc.bitcast(bi, jnp.float32)
                for _ in range(3):
                    rstd = rstd * (1.5 - 0.5 * xin * rstd * rstd)

                def col2_body(i, carry2):
                    gvec = gv[pl.ds(i * LANES, LANES)]
                    bvec = bv[pl.ds(i * LANES, LANES)]
                    for u in range(LANES):
                        col = i * LANES + u
                        colv = jnp.full((LANES,), col, jnp.int32)
                        e = et[col, :]
                        o = (e - mean) * rstd * gvec[u] + bvec[u]
                        plsc.store_scatter(rowsb, [rowv, colv], o)
                    return carry2
                lax.fori_loop(0, DIM // LANES, col2_body, 0)
                return carry
            lax.fori_loop(0, chp // LANES, group_body, 0)

        def step(c, b):
            wait_gather(idxs[b], rows[b], gsems[b])
            compute_chunk(rows[b], et)
            start_scatter(c, rows[b], ssems[b])
            g = c + 2
            b2 = (b + 2) % 3
            if isinstance(g, int) and g >= nchunk:
                return

            def issue():
                start_gather(g, idxs[b2], rows[b2], gsems[b2])

            if isinstance(g, int):
                if g >= 3:
                    wait_scatter(g - 3, rows[b2], ssems[b2])
                issue()
            else:
                @pl.when(g >= 3)
                def _():
                    wait_scatter(g - 3, rows[b2], ssems[b2])
                issue()

        start_gather(0, idx0, rows0, gsem0)
        start_gather(1, idx1, rows1, gsem1)

        nsteady = (nchunk - 2) // 3
        def outer(k, carry):
            c0 = 3 * k
            step(c0, 0)
            step(c0 + 1, 1)
            step(c0 + 2, 2)
            return carry
        lax.fori_loop(0, nsteady, outer, 0)
        for c in range(nsteady * 3, nchunk):
            step(c, c % 3)
        for c in range(nchunk - 3, nchunk):
            wait_scatter(c, rows[c % 3], ssems[c % 3])

    return pl.kernel(
        body,
        out_type=jax.ShapeDtypeStruct((tokens, DIM), jnp.float32),
        mesh=mesh,
        compiler_params=pltpu.CompilerParams(needs_layout_passes=False),
        scratch_types=[
            pltpu.VMEM((ch,), jnp.int32),
            pltpu.VMEM((ch,), jnp.int32),
            pltpu.VMEM((ch,), jnp.int32),
            pltpu.VMEM((chp, DIM), jnp.float32),
            pltpu.VMEM((chp, DIM), jnp.float32),
            pltpu.VMEM((chp, DIM), jnp.float32),
            pltpu.VMEM((DIM, chp), jnp.float32),
            pltpu.VMEM((DIM, LANES), jnp.float32),
            pltpu.VMEM((DIM,), jnp.float32),
            pltpu.VMEM((DIM,), jnp.float32),
            pltpu.VMEM((3 * LANES,), jnp.float32),
            pltpu.SemaphoreType.DMA,
            pltpu.SemaphoreType.DMA,
            pltpu.SemaphoreType.DMA,
            pltpu.SemaphoreType.DMA,
            pltpu.SemaphoreType.DMA,
            pltpu.SemaphoreType.DMA,
        ],
    )


def kernel(input_ids, weight, gamma, beta):
    batch, seq = input_ids.shape
    tokens = batch * seq
    pe = _positional_table(seq)

    wmin, wmax = _weight_minmax(weight)
    scale = (wmax - wmin) / 255.0
    zp = -128.0 - wmin / scale
    cst = jnp.concatenate([
        jnp.full((LANES,), 1.0 / scale, jnp.float32),
        jnp.full((LANES,), zp, jnp.float32),
        jnp.full((LANES,), scale, jnp.float32),
    ])

    ids_flat = input_ids.reshape(tokens).astype(jnp.int32)
    chp = (seq + LANES - 1) // LANES * LANES
    pe_t = jnp.pad(pe.T, ((0, 0), (0, chp - seq)))
    sc = _make_sc_kernel(tokens, seq)
    out = sc(ids_flat, weight, pe_t, gamma, beta, cst)
    return out.reshape(batch, seq, DIM)


# speedup vs baseline: 3.5456x; 3.5456x over previous
import numpy as np
import jax
import jax.numpy as jnp
from jax import lax
from jax.experimental import pallas as pl
from jax.experimental.pallas import tpu as pltpu
from jax.experimental.pallas import tpu_sc as plsc

VOCAB = 100000
DIM = 128
BASE = 512
NC, NS = 2, 16
NW = NC * NS
LANES = 16
NB = DIM // LANES
MAGIC = 12582912.0
S0, S1 = 96, 104


def _positional_table(seq_len):
    position = np.arange(BASE, dtype=np.float32)[:, None]
    div_term = np.exp(
        np.arange(0, DIM, 2, dtype=np.float32) * (-np.log(10000.0) / DIM))
    pe = np.zeros((BASE, DIM), dtype=np.float32)
    pe[:, 0::2] = np.sin(position * div_term)
    pe[:, 1::2] = np.cos(position * div_term)
    return jnp.asarray(pe[:seq_len])


def _minmax_body(w_ref, mn_ref, mx_ref):
    i = pl.program_id(0)
    bmn = jnp.min(w_ref[...])
    bmx = jnp.max(w_ref[...])

    @pl.when(i == 0)
    def _():
        mn_ref[0, 0] = bmn
        mx_ref[0, 0] = bmx

    @pl.when(i != 0)
    def _():
        mn_ref[0, 0] = jnp.minimum(mn_ref[0, 0], bmn)
        mx_ref[0, 0] = jnp.maximum(mx_ref[0, 0], bmx)


def _weight_minmax(weight):
    rows = 2000
    grid = VOCAB // rows
    mn, mx = pl.pallas_call(
        _minmax_body,
        grid=(grid,),
        in_specs=[pl.BlockSpec((rows, DIM), lambda i: (i, 0))],
        out_specs=[
            pl.BlockSpec((1, 1), lambda i: (0, 0), memory_space=pltpu.SMEM),
            pl.BlockSpec((1, 1), lambda i: (0, 0), memory_space=pltpu.SMEM),
        ],
        out_shape=[
            jax.ShapeDtypeStruct((1, 1), jnp.float32),
            jax.ShapeDtypeStruct((1, 1), jnp.float32),
        ],
    )(weight)
    return mn[0, 0], mx[0, 0]


def _rsqrt_scalar(x):
    i = lax.bitcast_convert_type(x, jnp.int32)
    i = jnp.int32(0x5F3759DF) - (i >> 1)
    y = lax.bitcast_convert_type(i, jnp.float32)
    for _ in range(3):
        y = y * (1.5 - 0.5 * x * y * y)
    return y


def _make_sc_kernel(tokens, seq):
    tok_w = tokens // NW
    ch = seq
    chp = (ch + LANES - 1) // LANES * LANES
    nchunk = tok_w // ch
    assert tok_w % ch == 0 and tokens % NW == 0 and nchunk >= 3

    mesh = plsc.VectorSubcoreMesh(core_axis_name="c", subcore_axis_name="s")

    def body(ids_hbm, w_hbm, pe_hbm, gam_hbm, bet_hbm, cst_hbm, out_hbm,
             idx0, idx1, idx2, rows0, rows1, rows2, pev, gv, bv, cv,
             gsem0, gsem1, gsem2, ssem0, ssem1, ssem2):
        idxs = (idx0, idx1, idx2)
        rows = (rows0, rows1, rows2)
        gsems = (gsem0, gsem1, gsem2)
        ssems = (ssem0, ssem1, ssem2)

        cid = lax.axis_index("c")
        sid = lax.axis_index("s")
        wid = sid * NC + cid
        tok0 = wid * tok_w

        pltpu.sync_copy(pe_hbm, pev)
        pltpu.sync_copy(gam_hbm, gv)
        pltpu.sync_copy(bet_hbm, bv)
        pltpu.sync_copy(cst_hbm, cv)

        inv = cv[pl.ds(0, LANES)]
        zp = cv[pl.ds(LANES, LANES)]
        scale = cv[pl.ds(2 * LANES, LANES)]
        zps = zp * scale

        def pe_body(r, carry):
            for j in range(NB):
                sl = pl.ds(j * LANES, LANES)
                pev[r, sl] = pev[r, sl] - zps
            return carry
        lax.fori_loop(0, seq, pe_body, 0)

        def start_gather(c, idxb, rowsb, gsem):
            t0 = tok0 + c * ch
            pltpu.sync_copy(ids_hbm.at[pl.ds(t0, ch)], idxb)
            pltpu.async_copy(w_hbm.at[idxb.at[pl.ds(0, S0)]],
                             rowsb.at[pl.ds(0, S0)], gsem)
            pltpu.async_copy(w_hbm.at[idxb.at[pl.ds(S0, S1)]],
                             rowsb.at[pl.ds(S0, S1)], gsem)

        def wait_gather(idxb, rowsb, gsem):
            pltpu.make_async_copy(w_hbm.at[idxb.at[pl.ds(0, S0)]],
                                  rowsb.at[pl.ds(0, S0)], gsem).wait()
            pltpu.make_async_copy(w_hbm.at[idxb.at[pl.ds(S0, S1)]],
                                  rowsb.at[pl.ds(S0, S1)], gsem).wait()

        def start_scatter(c, rowsb, ssem):
            t0 = tok0 + c * ch
            pltpu.async_copy(rowsb.at[pl.ds(0, ch)], out_hbm.at[pl.ds(t0, ch)],
                             ssem)

        def wait_scatter(c, rowsb, ssem):
            t0 = tok0 + c * ch
            pltpu.make_async_copy(rowsb.at[pl.ds(0, ch)],
                                  out_hbm.at[pl.ds(t0, ch)], ssem).wait()

        RPI = 4
        iota = lax.iota(jnp.int32, LANES)
        perms = [lax.bitwise_xor(iota, jnp.int32(k)) for k in (1, 2, 4, 8)]

        def allsum(v):
            for p in perms:
                v = v + v.at[p].get(mode="promise_in_bounds")
            return v

        def vrsqrt(x):
            bi = plsc.bitcast(x, jnp.int32)
            bi = jnp.full((LANES,), 0x5F3759DF, jnp.int32) - (bi >> 1)
            y = plsc.bitcast(bi, jnp.float32)
            for _ in range(3):
                y = y * (1.5 - 0.5 * x * y * y)
            return y

        def compute_chunk(rowsb):
            def row_body(it, carry):
                r0 = it * RPI
                ess, means, rstds = [], [], []
                for k in range(RPI):
                    r = r0 + k
                    es = []
                    for j in range(NB):
                        sl = pl.ds(j * LANES, LANES)
                        x = rowsb[r, sl]
                        y = x * inv + zp
                        q = (y + MAGIC) - MAGIC
                        es.append(q * scale + pev[r, sl])
                    vs = es[0]
                    vq = es[0] * es[0]
                    for j in range(1, NB):
                        vs = vs + es[j]
                        vq = vq + es[j] * es[j]
                    mean = allsum(vs) * (1.0 / DIM)
                    var = allsum(vq) * (1.0 / DIM) - mean * mean
                    ess.append(es)
                    means.append(mean)
                    rstds.append(vrsqrt(var + 1e-5))
                for k in range(RPI):
                    r = r0 + k
                    for j in range(NB):
                        sl = pl.ds(j * LANES, LANES)
                        a = gv[sl] * rstds[k]
                        rowsb[r, sl] = (ess[k][j] - means[k]) * a + bv[sl]
                return carry
            lax.fori_loop(0, ch // RPI, row_body, 0)

        def step(c, b):
            wait_gather(idxs[b], rows[b], gsems[b])
            compute_chunk(rows[b])
            start_scatter(c, rows[b], ssems[b])
            g = c + 2
            b2 = (b + 2) % 3
            if isinstance(g, int) and g >= nchunk:
                return

            def issue():
                start_gather(g, idxs[b2], rows[b2], gsems[b2])

            if isinstance(g, int):
                if g >= 3:
                    wait_scatter(g - 3, rows[b2], ssems[b2])
                issue()
            else:
                @pl.when(g >= 3)
                def _():
                    wait_scatter(g - 3, rows[b2], ssems[b2])
                issue()

        start_gather(0, idx0, rows0, gsem0)
        start_gather(1, idx1, rows1, gsem1)

        nsteady = (nchunk - 2) // 3
        def outer(k, carry):
            c0 = 3 * k
            step(c0, 0)
            step(c0 + 1, 1)
            step(c0 + 2, 2)
            return carry
        lax.fori_loop(0, nsteady, outer, 0)
        for c in range(nsteady * 3, nchunk):
            step(c, c % 3)
        for c in range(nchunk - 3, nchunk):
            wait_scatter(c, rows[c % 3], ssems[c % 3])

    return pl.kernel(
        body,
        out_type=jax.ShapeDtypeStruct((tokens, DIM), jnp.float32),
        mesh=mesh,
        compiler_params=pltpu.CompilerParams(needs_layout_passes=False),
        scratch_types=[
            pltpu.VMEM((ch,), jnp.int32),
            pltpu.VMEM((ch,), jnp.int32),
            pltpu.VMEM((ch,), jnp.int32),
            pltpu.VMEM((ch, DIM), jnp.float32),
            pltpu.VMEM((ch, DIM), jnp.float32),
            pltpu.VMEM((ch, DIM), jnp.float32),
            pltpu.VMEM((seq, DIM), jnp.float32),
            pltpu.VMEM((DIM,), jnp.float32),
            pltpu.VMEM((DIM,), jnp.float32),
            pltpu.VMEM((3 * LANES,), jnp.float32),
            pltpu.SemaphoreType.DMA,
            pltpu.SemaphoreType.DMA,
            pltpu.SemaphoreType.DMA,
            pltpu.SemaphoreType.DMA,
            pltpu.SemaphoreType.DMA,
            pltpu.SemaphoreType.DMA,
        ],
    )


def kernel(input_ids, weight, gamma, beta):
    batch, seq = input_ids.shape
    tokens = batch * seq
    pe = _positional_table(seq)

    wmin, wmax = _weight_minmax(weight)
    scale = (wmax - wmin) / 255.0
    zp = -128.0 - wmin / scale
    cst = jnp.concatenate([
        jnp.full((LANES,), 1.0 / scale, jnp.float32),
        jnp.full((LANES,), zp, jnp.float32),
        jnp.full((LANES,), scale, jnp.float32),
    ])

    ids_flat = input_ids.reshape(tokens).astype(jnp.int32)
    sc = _make_sc_kernel(tokens, seq)
    out = sc(ids_flat, weight, pe, gamma, beta, cst)
    return out.reshape(batch, seq, DIM)


# speedup vs baseline: 7.4725x; 2.1075x over previous
import numpy as np
import jax
import jax.numpy as jnp
from jax import lax
from jax.experimental import pallas as pl
from jax.experimental.pallas import tpu as pltpu
from jax.experimental.pallas import tpu_sc as plsc

VOCAB = 100000
DIM = 128
BASE = 512
NC, NS = 2, 16
NW = NC * NS
LANES = 16
GCH = 256
SEQ_PER_BLK = 8


def _positional_table(seq_len):
    position = np.arange(BASE, dtype=np.float32)[:, None]
    div_term = np.exp(
        np.arange(0, DIM, 2, dtype=np.float32) * (-np.log(10000.0) / DIM))
    pe = np.zeros((BASE, DIM), dtype=np.float32)
    pe[:, 0::2] = np.sin(position * div_term)
    pe[:, 1::2] = np.cos(position * div_term)
    return jnp.asarray(pe[:seq_len])


def _minmax_body(w_ref, mn_ref, mx_ref):
    i = pl.program_id(0)
    bmn = jnp.min(w_ref[...])
    bmx = jnp.max(w_ref[...])

    @pl.when(i == 0)
    def _():
        mn_ref[0, 0] = bmn
        mx_ref[0, 0] = bmx

    @pl.when(i != 0)
    def _():
        mn_ref[0, 0] = jnp.minimum(mn_ref[0, 0], bmn)
        mx_ref[0, 0] = jnp.maximum(mx_ref[0, 0], bmx)


def _weight_minmax(weight):
    rows = 2000
    grid = VOCAB // rows
    mn, mx = pl.pallas_call(
        _minmax_body,
        grid=(grid,),
        in_specs=[pl.BlockSpec((rows, DIM), lambda i: (i, 0))],
        out_specs=[
            pl.BlockSpec((1, 1), lambda i: (0, 0), memory_space=pltpu.SMEM),
            pl.BlockSpec((1, 1), lambda i: (0, 0), memory_space=pltpu.SMEM),
        ],
        out_shape=[
            jax.ShapeDtypeStruct((1, 1), jnp.float32),
            jax.ShapeDtypeStruct((1, 1), jnp.float32),
        ],
    )(weight)
    return mn[0, 0], mx[0, 0]


def _make_sc_gather(tokens):
    tok_w = tokens // NW
    nchunk = tok_w // GCH
    assert tokens % NW == 0 and tok_w % GCH == 0 and nchunk >= 3
    half = GCH // 2

    mesh = plsc.VectorSubcoreMesh(core_axis_name="c", subcore_axis_name="s")

    def body(ids_hbm, w_hbm, emb_hbm,
             idx0, idx1, idx2, rows0, rows1, rows2,
             gsem0, gsem1, gsem2, ssem0, ssem1, ssem2):
        idxs = (idx0, idx1, idx2)
        rows = (rows0, rows1, rows2)
        gsems = (gsem0, gsem1, gsem2)
        ssems = (ssem0, ssem1, ssem2)

        cid = lax.axis_index("c")
        sid = lax.axis_index("s")
        wid = sid * NC + cid
        tok0 = wid * tok_w

        def start_gather(c, idxb, rowsb, gsem):
            t0 = tok0 + c * GCH
            pltpu.sync_copy(ids_hbm.at[pl.ds(t0, GCH)], idxb)
            pltpu.async_copy(w_hbm.at[idxb.at[pl.ds(0, half)]],
                             rowsb.at[pl.ds(0, half)], gsem)
            pltpu.async_copy(w_hbm.at[idxb.at[pl.ds(half, half)]],
                             rowsb.at[pl.ds(half, half)], gsem)

        def wait_gather(idxb, rowsb, gsem):
            pltpu.make_async_copy(w_hbm.at[idxb.at[pl.ds(0, half)]],
                                  rowsb.at[pl.ds(0, half)], gsem).wait()
            pltpu.make_async_copy(w_hbm.at[idxb.at[pl.ds(half, half)]],
                                  rowsb.at[pl.ds(half, half)], gsem).wait()

        def start_scatter(c, rowsb, ssem):
            t0 = tok0 + c * GCH
            pltpu.async_copy(rowsb, emb_hbm.at[pl.ds(t0, GCH)], ssem)

        def wait_scatter(c, rowsb, ssem):
            t0 = tok0 + c * GCH
            pltpu.make_async_copy(rowsb, emb_hbm.at[pl.ds(t0, GCH)],
                                  ssem).wait()

        def step(c, b):
            wait_gather(idxs[b], rows[b], gsems[b])
            start_scatter(c, rows[b], ssems[b])
            g = c + 2
            b2 = (b + 2) % 3
            if isinstance(g, int) and g >= nchunk:
                return

            def issue():
                start_gather(g, idxs[b2], rows[b2], gsems[b2])

            if isinstance(g, int):
                if g >= 3:
                    wait_scatter(g - 3, rows[b2], ssems[b2])
                issue()
            else:
                @pl.when(g >= 3)
                def _():
                    wait_scatter(g - 3, rows[b2], ssems[b2])
                issue()

        start_gather(0, idx0, rows0, gsem0)
        start_gather(1, idx1, rows1, gsem1)

        nsteady = (nchunk - 2) // 3
        def outer(k, carry):
            c0 = 3 * k
            step(c0, 0)
            step(c0 + 1, 1)
            step(c0 + 2, 2)
            return carry
        lax.fori_loop(0, nsteady, outer, 0)
        for c in range(nsteady * 3, nchunk):
            step(c, c % 3)
        for c in range(nchunk - 3, nchunk):
            wait_scatter(c, rows[c % 3], ssems[c % 3])

    return pl.kernel(
        body,
        out_type=jax.ShapeDtypeStruct((tokens, DIM), jnp.float32),
        mesh=mesh,
        scratch_types=[
            pltpu.VMEM((GCH,), jnp.int32),
            pltpu.VMEM((GCH,), jnp.int32),
            pltpu.VMEM((GCH,), jnp.int32),
            pltpu.VMEM((GCH, DIM), jnp.float32),
            pltpu.VMEM((GCH, DIM), jnp.float32),
            pltpu.VMEM((GCH, DIM), jnp.float32),
            pltpu.SemaphoreType.DMA,
            pltpu.SemaphoreType.DMA,
            pltpu.SemaphoreType.DMA,
            pltpu.SemaphoreType.DMA,
            pltpu.SemaphoreType.DMA,
            pltpu.SemaphoreType.DMA,
        ],
    )


def _ln_body(emb_ref, pe_ref, cst_ref, gam_ref, bet_ref, out_ref):
    inv = cst_ref[0, 0]
    zp = cst_ref[0, 1]
    scale = cst_ref[0, 2]
    x = emb_ref[...]
    q = jnp.round(x * inv + zp)
    e = q * scale + pe_ref[...]
    mean = jnp.mean(e, axis=-1, keepdims=True)
    var = jnp.mean(e * e, axis=-1, keepdims=True) - mean * mean
    r = lax.rsqrt(var + 1e-5)
    out_ref[...] = (e - mean) * r * gam_ref[...] + bet_ref[...]


def _ln_pass(emb, pe_blk, cst, gamma, beta, tokens, seq):
    blk = seq * SEQ_PER_BLK
    grid = tokens // blk
    assert tokens % blk == 0
    return pl.pallas_call(
        _ln_body,
        grid=(grid,),
        in_specs=[
            pl.BlockSpec((blk, DIM), lambda i: (i, 0)),
            pl.BlockSpec((blk, DIM), lambda i: (0, 0)),
            pl.BlockSpec((1, 3), lambda i: (0, 0), memory_space=pltpu.SMEM),
            pl.BlockSpec((1, DIM), lambda i: (0, 0)),
            pl.BlockSpec((1, DIM), lambda i: (0, 0)),
        ],
        out_specs=pl.BlockSpec((blk, DIM), lambda i: (i, 0)),
        out_shape=jax.ShapeDtypeStruct((tokens, DIM), jnp.float32),
    )(emb, pe_blk, cst, gamma, beta)


def kernel(input_ids, weight, gamma, beta):
    batch, seq = input_ids.shape
    tokens = batch * seq
    pe = _positional_table(seq)

    wmin, wmax = _weight_minmax(weight)
    scale = (wmax - wmin) / 255.0
    zp = -128.0 - wmin / scale
    cst = jnp.stack([1.0 / scale, zp, scale]).reshape(1, 3)
    pe_blk = jnp.tile(pe, (SEQ_PER_BLK, 1)) - zp * scale

    ids_flat = input_ids.reshape(tokens).astype(jnp.int32)
    gather = _make_sc_gather(tokens)
    emb = gather(ids_flat, weight)
    out = _ln_pass(emb, pe_blk, cst, gamma.reshape(1, DIM),
                   beta.reshape(1, DIM), tokens, seq)
    return out.reshape(batch, seq, DIM)


# speedup vs baseline: 8.6424x; 1.1566x over previous
import numpy as np
import jax
import jax.numpy as jnp
from jax import lax
from jax.experimental import pallas as pl
from jax.experimental.pallas import tpu as pltpu
from jax.experimental.pallas import tpu_sc as plsc

VOCAB = 100000
DIM = 128
BASE = 512
NC, NS = 2, 16
NW = NC * NS
LANES = 16
GCH = 256
SEQ_PER_BLK = 16


def _positional_table(seq_len):
    position = np.arange(BASE, dtype=np.float32)[:, None]
    div_term = np.exp(
        np.arange(0, DIM, 2, dtype=np.float32) * (-np.log(10000.0) / DIM))
    pe = np.zeros((BASE, DIM), dtype=np.float32)
    pe[:, 0::2] = np.sin(position * div_term)
    pe[:, 1::2] = np.cos(position * div_term)
    return jnp.asarray(pe[:seq_len])


def _minmax_body(w_ref, mn_ref, mx_ref):
    i = pl.program_id(0)
    bmn = jnp.min(w_ref[...])
    bmx = jnp.max(w_ref[...])

    @pl.when(i == 0)
    def _():
        mn_ref[0, 0] = bmn
        mx_ref[0, 0] = bmx

    @pl.when(i != 0)
    def _():
        mn_ref[0, 0] = jnp.minimum(mn_ref[0, 0], bmn)
        mx_ref[0, 0] = jnp.maximum(mx_ref[0, 0], bmx)


def _weight_minmax(weight):
    rows = 2000
    grid = VOCAB // rows
    mn, mx = pl.pallas_call(
        _minmax_body,
        grid=(grid,),
        in_specs=[pl.BlockSpec((rows, DIM), lambda i: (i, 0))],
        out_specs=[
            pl.BlockSpec((1, 1), lambda i: (0, 0), memory_space=pltpu.SMEM),
            pl.BlockSpec((1, 1), lambda i: (0, 0), memory_space=pltpu.SMEM),
        ],
        out_shape=[
            jax.ShapeDtypeStruct((1, 1), jnp.float32),
            jax.ShapeDtypeStruct((1, 1), jnp.float32),
        ],
    )(weight)
    return mn[0, 0], mx[0, 0]


def _make_sc_gather(tokens):
    tok_w = tokens // NW
    nchunk = tok_w // GCH
    assert tokens % NW == 0 and tok_w % GCH == 0 and nchunk >= 3
    half = GCH // 2

    mesh = plsc.VectorSubcoreMesh(core_axis_name="c", subcore_axis_name="s")

    def body(ids_hbm, w_hbm, emb_hbm,
             idx0, idx1, idx2, rows0, rows1, rows2,
             gsem0, gsem1, gsem2, ssem0, ssem1, ssem2):
        idxs = (idx0, idx1, idx2)
        rows = (rows0, rows1, rows2)
        gsems = (gsem0, gsem1, gsem2)
        ssems = (ssem0, ssem1, ssem2)

        cid = lax.axis_index("c")
        sid = lax.axis_index("s")
        wid = sid * NC + cid
        tok0 = wid * tok_w

        def start_gather(c, idxb, rowsb, gsem):
            t0 = tok0 + c * GCH
            pltpu.sync_copy(ids_hbm.at[pl.ds(t0, GCH)], idxb)
            pltpu.async_copy(w_hbm.at[idxb.at[pl.ds(0, half)]],
                             rowsb.at[pl.ds(0, half)], gsem)
            pltpu.async_copy(w_hbm.at[idxb.at[pl.ds(half, half)]],
                             rowsb.at[pl.ds(half, half)], gsem)

        def wait_gather(idxb, rowsb, gsem):
            pltpu.make_async_copy(w_hbm.at[idxb.at[pl.ds(0, half)]],
                                  rowsb.at[pl.ds(0, half)], gsem).wait()
            pltpu.make_async_copy(w_hbm.at[idxb.at[pl.ds(half, half)]],
                                  rowsb.at[pl.ds(half, half)], gsem).wait()

        def start_scatter(c, rowsb, ssem):
            t0 = tok0 + c * GCH
            pltpu.async_copy(rowsb, emb_hbm.at[pl.ds(t0, GCH)], ssem)

        def wait_scatter(c, rowsb, ssem):
            t0 = tok0 + c * GCH
            pltpu.make_async_copy(rowsb, emb_hbm.at[pl.ds(t0, GCH)],
                                  ssem).wait()

        def step(c, b):
            wait_gather(idxs[b], rows[b], gsems[b])
            start_scatter(c, rows[b], ssems[b])
            g = c + 2
            b2 = (b + 2) % 3
            if isinstance(g, int) and g >= nchunk:
                return

            def issue():
                start_gather(g, idxs[b2], rows[b2], gsems[b2])

            if isinstance(g, int):
                if g >= 3:
                    wait_scatter(g - 3, rows[b2], ssems[b2])
                issue()
            else:
                @pl.when(g >= 3)
                def _():
                    wait_scatter(g - 3, rows[b2], ssems[b2])
                issue()

        start_gather(0, idx0, rows0, gsem0)
        start_gather(1, idx1, rows1, gsem1)

        nsteady = (nchunk - 2) // 3
        def outer(k, carry):
            c0 = 3 * k
            step(c0, 0)
            step(c0 + 1, 1)
            step(c0 + 2, 2)
            return carry
        lax.fori_loop(0, nsteady, outer, 0)
        for c in range(nsteady * 3, nchunk):
            step(c, c % 3)
        for c in range(nchunk - 3, nchunk):
            wait_scatter(c, rows[c % 3], ssems[c % 3])

    return pl.kernel(
        body,
        out_type=jax.ShapeDtypeStruct((tokens, DIM), jnp.float32),
        mesh=mesh,
        scratch_types=[
            pltpu.VMEM((GCH,), jnp.int32),
            pltpu.VMEM((GCH,), jnp.int32),
            pltpu.VMEM((GCH,), jnp.int32),
            pltpu.VMEM((GCH, DIM), jnp.float32),
            pltpu.VMEM((GCH, DIM), jnp.float32),
            pltpu.VMEM((GCH, DIM), jnp.float32),
            pltpu.SemaphoreType.DMA,
            pltpu.SemaphoreType.DMA,
            pltpu.SemaphoreType.DMA,
            pltpu.SemaphoreType.DMA,
            pltpu.SemaphoreType.DMA,
            pltpu.SemaphoreType.DMA,
        ],
    )


def _ln_body(emb_ref, pe_ref, cst_ref, gam_ref, bet_ref, out_ref):
    inv = cst_ref[0, 0]
    zp = cst_ref[0, 1]
    scale = cst_ref[0, 2]
    x = emb_ref[...]
    q = jnp.round(x * inv + zp)
    e = q * scale + pe_ref[...]
    mean = jnp.mean(e, axis=-1, keepdims=True)
    var = jnp.mean(e * e, axis=-1, keepdims=True) - mean * mean
    r = lax.rsqrt(var + 1e-5)
    out_ref[...] = (e - mean) * r * gam_ref[...] + bet_ref[...]


def _ln_pass(emb, pe_blk, cst, gamma, beta, tokens, seq):
    blk = seq * SEQ_PER_BLK
    grid = tokens // blk
    assert tokens % blk == 0
    return pl.pallas_call(
        _ln_body,
        grid=(grid,),
        in_specs=[
            pl.BlockSpec((blk, DIM), lambda i: (i, 0)),
            pl.BlockSpec((blk, DIM), lambda i: (0, 0)),
            pl.BlockSpec((1, 3), lambda i: (0, 0), memory_space=pltpu.SMEM),
            pl.BlockSpec((1, DIM), lambda i: (0, 0)),
            pl.BlockSpec((1, DIM), lambda i: (0, 0)),
        ],
        out_specs=pl.BlockSpec((blk, DIM), lambda i: (i, 0)),
        out_shape=jax.ShapeDtypeStruct((tokens, DIM), jnp.float32),
    )(emb, pe_blk, cst, gamma, beta)


def kernel(input_ids, weight, gamma, beta):
    batch, seq = input_ids.shape
    tokens = batch * seq
    pe = _positional_table(seq)

    wmin, wmax = _weight_minmax(weight)
    scale = (wmax - wmin) / 255.0
    zp = -128.0 - wmin / scale
    cst = jnp.stack([1.0 / scale, zp, scale]).reshape(1, 3)
    pe_blk = jnp.tile(pe, (SEQ_PER_BLK, 1)) - zp * scale

    ids_flat = input_ids.reshape(tokens).astype(jnp.int32)
    gather = _make_sc_gather(tokens)
    emb = gather(ids_flat, weight)
    out = _ln_pass(emb, pe_blk, cst, gamma.reshape(1, DIM),
                   beta.reshape(1, DIM), tokens, seq)
    return out.reshape(batch, seq, DIM)


# speedup vs baseline: 9.4086x; 1.0887x over previous
import numpy as np
import jax
import jax.numpy as jnp
from jax import lax
from jax.experimental import pallas as pl
from jax.experimental.pallas import tpu as pltpu
from jax.experimental.pallas import tpu_sc as plsc

VOCAB = 100000
DIM = 128
BASE = 512
NC, NS = 2, 16
NW = NC * NS
LANES = 16
GCH = 256
SEQ_PER_BLK = 32


def _positional_table(seq_len):
    position = np.arange(BASE, dtype=np.float32)[:, None]
    div_term = np.exp(
        np.arange(0, DIM, 2, dtype=np.float32) * (-np.log(10000.0) / DIM))
    pe = np.zeros((BASE, DIM), dtype=np.float32)
    pe[:, 0::2] = np.sin(position * div_term)
    pe[:, 1::2] = np.cos(position * div_term)
    return jnp.asarray(pe[:seq_len])


def _minmax_body(w_ref, mn_ref, mx_ref):
    i = pl.program_id(0)
    bmn = jnp.min(w_ref[...])
    bmx = jnp.max(w_ref[...])

    @pl.when(i == 0)
    def _():
        mn_ref[0, 0] = bmn
        mx_ref[0, 0] = bmx

    @pl.when(i != 0)
    def _():
        mn_ref[0, 0] = jnp.minimum(mn_ref[0, 0], bmn)
        mx_ref[0, 0] = jnp.maximum(mx_ref[0, 0], bmx)


def _weight_minmax(weight):
    rows = 2000
    grid = VOCAB // rows
    mn, mx = pl.pallas_call(
        _minmax_body,
        grid=(grid,),
        in_specs=[pl.BlockSpec((rows, DIM), lambda i: (i, 0))],
        out_specs=[
            pl.BlockSpec((1, 1), lambda i: (0, 0), memory_space=pltpu.SMEM),
            pl.BlockSpec((1, 1), lambda i: (0, 0), memory_space=pltpu.SMEM),
        ],
        out_shape=[
            jax.ShapeDtypeStruct((1, 1), jnp.float32),
            jax.ShapeDtypeStruct((1, 1), jnp.float32),
        ],
    )(weight)
    return mn[0, 0], mx[0, 0]


def _make_sc_gather(tokens):
    tok_w = tokens // NW
    nchunk = tok_w // GCH
    assert tokens % NW == 0 and tok_w % GCH == 0 and nchunk >= 3
    half = GCH // 2

    mesh = plsc.VectorSubcoreMesh(core_axis_name="c", subcore_axis_name="s")

    def body(ids_hbm, w_hbm, emb_hbm,
             idx0, idx1, idx2, rows0, rows1, rows2,
             gsem0, gsem1, gsem2, ssem0, ssem1, ssem2):
        idxs = (idx0, idx1, idx2)
        rows = (rows0, rows1, rows2)
        gsems = (gsem0, gsem1, gsem2)
        ssems = (ssem0, ssem1, ssem2)

        cid = lax.axis_index("c")
        sid = lax.axis_index("s")
        wid = sid * NC + cid
        tok0 = wid * tok_w

        def start_gather(c, idxb, rowsb, gsem):
            t0 = tok0 + c * GCH
            pltpu.sync_copy(ids_hbm.at[pl.ds(t0, GCH)], idxb)
            pltpu.async_copy(w_hbm.at[idxb.at[pl.ds(0, half)]],
                             rowsb.at[pl.ds(0, half)], gsem)
            pltpu.async_copy(w_hbm.at[idxb.at[pl.ds(half, half)]],
                             rowsb.at[pl.ds(half, half)], gsem)

        def wait_gather(idxb, rowsb, gsem):
            pltpu.make_async_copy(w_hbm.at[idxb.at[pl.ds(0, half)]],
                                  rowsb.at[pl.ds(0, half)], gsem).wait()
            pltpu.make_async_copy(w_hbm.at[idxb.at[pl.ds(half, half)]],
                                  rowsb.at[pl.ds(half, half)], gsem).wait()

        def start_scatter(c, rowsb, ssem):
            t0 = tok0 + c * GCH
            pltpu.async_copy(rowsb, emb_hbm.at[pl.ds(t0, GCH)], ssem)

        def wait_scatter(c, rowsb, ssem):
            t0 = tok0 + c * GCH
            pltpu.make_async_copy(rowsb, emb_hbm.at[pl.ds(t0, GCH)],
                                  ssem).wait()

        def step(c, b):
            wait_gather(idxs[b], rows[b], gsems[b])
            start_scatter(c, rows[b], ssems[b])
            g = c + 2
            b2 = (b + 2) % 3
            if isinstance(g, int) and g >= nchunk:
                return

            def issue():
                start_gather(g, idxs[b2], rows[b2], gsems[b2])

            if isinstance(g, int):
                if g >= 3:
                    wait_scatter(g - 3, rows[b2], ssems[b2])
                issue()
            else:
                @pl.when(g >= 3)
                def _():
                    wait_scatter(g - 3, rows[b2], ssems[b2])
                issue()

        start_gather(0, idx0, rows0, gsem0)
        start_gather(1, idx1, rows1, gsem1)

        nsteady = (nchunk - 2) // 3
        def outer(k, carry):
            c0 = 3 * k
            step(c0, 0)
            step(c0 + 1, 1)
            step(c0 + 2, 2)
            return carry
        lax.fori_loop(0, nsteady, outer, 0)
        for c in range(nsteady * 3, nchunk):
            step(c, c % 3)
        for c in range(nchunk - 3, nchunk):
            wait_scatter(c, rows[c % 3], ssems[c % 3])

    return pl.kernel(
        body,
        out_type=jax.ShapeDtypeStruct((tokens, DIM), jnp.float32),
        mesh=mesh,
        scratch_types=[
            pltpu.VMEM((GCH,), jnp.int32),
            pltpu.VMEM((GCH,), jnp.int32),
            pltpu.VMEM((GCH,), jnp.int32),
            pltpu.VMEM((GCH, DIM), jnp.float32),
            pltpu.VMEM((GCH, DIM), jnp.float32),
            pltpu.VMEM((GCH, DIM), jnp.float32),
            pltpu.SemaphoreType.DMA,
            pltpu.SemaphoreType.DMA,
            pltpu.SemaphoreType.DMA,
            pltpu.SemaphoreType.DMA,
            pltpu.SemaphoreType.DMA,
            pltpu.SemaphoreType.DMA,
        ],
    )


def _ln_body(emb_ref, pe_ref, cst_ref, gam_ref, bet_ref, out_ref):
    inv = cst_ref[0, 0]
    zp = cst_ref[0, 1]
    scale = cst_ref[0, 2]
    x = emb_ref[...]
    q = jnp.round(x * inv + zp)
    e = q * scale + pe_ref[...]
    mean = jnp.mean(e, axis=-1, keepdims=True)
    var = jnp.mean(e * e, axis=-1, keepdims=True) - mean * mean
    r = lax.rsqrt(var + 1e-5)
    out_ref[...] = (e - mean) * r * gam_ref[...] + bet_ref[...]


def _ln_pass(emb, pe_blk, cst, gamma, beta, tokens, seq):
    blk = seq * SEQ_PER_BLK
    grid = tokens // blk
    assert tokens % blk == 0
    return pl.pallas_call(
        _ln_body,
        grid=(grid,),
        in_specs=[
            pl.BlockSpec((blk, DIM), lambda i: (i, 0)),
            pl.BlockSpec((blk, DIM), lambda i: (0, 0)),
            pl.BlockSpec((1, 3), lambda i: (0, 0), memory_space=pltpu.SMEM),
            pl.BlockSpec((1, DIM), lambda i: (0, 0)),
            pl.BlockSpec((1, DIM), lambda i: (0, 0)),
        ],
        out_specs=pl.BlockSpec((blk, DIM), lambda i: (i, 0)),
        out_shape=jax.ShapeDtypeStruct((tokens, DIM), jnp.float32),
    )(emb, pe_blk, cst, gamma, beta)


def kernel(input_ids, weight, gamma, beta):
    batch, seq = input_ids.shape
    tokens = batch * seq
    pe = _positional_table(seq)

    wmin, wmax = _weight_minmax(weight)
    scale = (wmax - wmin) / 255.0
    zp = -128.0 - wmin / scale
    cst = jnp.stack([1.0 / scale, zp, scale]).reshape(1, 3)
    pe_blk = jnp.tile(pe, (SEQ_PER_BLK, 1)) - zp * scale

    ids_flat = input_ids.reshape(tokens).astype(jnp.int32)
    gather = _make_sc_gather(tokens)
    emb = gather(ids_flat, weight)
    out = _ln_pass(emb, pe_blk, cst, gamma.reshape(1, DIM),
                   beta.reshape(1, DIM), tokens, seq)
    return out.reshape(batch, seq, DIM)


# speedup vs baseline: 9.7370x; 1.0349x over previous
import numpy as np
import jax
import jax.numpy as jnp
from jax import lax
from jax.experimental import pallas as pl
from jax.experimental.pallas import tpu as pltpu
from jax.experimental.pallas import tpu_sc as plsc

VOCAB = 100000
DIM = 128
BASE = 512
NC, NS = 2, 16
NW = NC * NS
LANES = 16
GCH = 256
SEQ_PER_BLK = 64


def _positional_table(seq_len):
    position = np.arange(BASE, dtype=np.float32)[:, None]
    div_term = np.exp(
        np.arange(0, DIM, 2, dtype=np.float32) * (-np.log(10000.0) / DIM))
    pe = np.zeros((BASE, DIM), dtype=np.float32)
    pe[:, 0::2] = np.sin(position * div_term)
    pe[:, 1::2] = np.cos(position * div_term)
    return jnp.asarray(pe[:seq_len])


def _minmax_body(w_ref, mn_ref, mx_ref):
    i = pl.program_id(0)
    bmn = jnp.min(w_ref[...])
    bmx = jnp.max(w_ref[...])

    @pl.when(i == 0)
    def _():
        mn_ref[0, 0] = bmn
        mx_ref[0, 0] = bmx

    @pl.when(i != 0)
    def _():
        mn_ref[0, 0] = jnp.minimum(mn_ref[0, 0], bmn)
        mx_ref[0, 0] = jnp.maximum(mx_ref[0, 0], bmx)


def _weight_minmax(weight):
    rows = 2000
    grid = VOCAB // rows
    mn, mx = pl.pallas_call(
        _minmax_body,
        grid=(grid,),
        in_specs=[pl.BlockSpec((rows, DIM), lambda i: (i, 0))],
        out_specs=[
            pl.BlockSpec((1, 1), lambda i: (0, 0), memory_space=pltpu.SMEM),
            pl.BlockSpec((1, 1), lambda i: (0, 0), memory_space=pltpu.SMEM),
        ],
        out_shape=[
            jax.ShapeDtypeStruct((1, 1), jnp.float32),
            jax.ShapeDtypeStruct((1, 1), jnp.float32),
        ],
    )(weight)
    return mn[0, 0], mx[0, 0]


def _make_sc_gather(tokens):
    tok_w = tokens // NW
    nchunk = tok_w // GCH
    assert tokens % NW == 0 and tok_w % GCH == 0 and nchunk >= 3
    half = GCH // 2

    mesh = plsc.VectorSubcoreMesh(core_axis_name="c", subcore_axis_name="s")

    def body(ids_hbm, w_hbm, emb_hbm,
             idx0, idx1, idx2, rows0, rows1, rows2,
             gsem0, gsem1, gsem2, ssem0, ssem1, ssem2):
        idxs = (idx0, idx1, idx2)
        rows = (rows0, rows1, rows2)
        gsems = (gsem0, gsem1, gsem2)
        ssems = (ssem0, ssem1, ssem2)

        cid = lax.axis_index("c")
        sid = lax.axis_index("s")
        wid = sid * NC + cid
        tok0 = wid * tok_w

        def start_gather(c, idxb, rowsb, gsem):
            t0 = tok0 + c * GCH
            pltpu.sync_copy(ids_hbm.at[pl.ds(t0, GCH)], idxb)
            pltpu.async_copy(w_hbm.at[idxb.at[pl.ds(0, half)]],
                             rowsb.at[pl.ds(0, half)], gsem)
            pltpu.async_copy(w_hbm.at[idxb.at[pl.ds(half, half)]],
                             rowsb.at[pl.ds(half, half)], gsem)

        def wait_gather(idxb, rowsb, gsem):
            pltpu.make_async_copy(w_hbm.at[idxb.at[pl.ds(0, half)]],
                                  rowsb.at[pl.ds(0, half)], gsem).wait()
            pltpu.make_async_copy(w_hbm.at[idxb.at[pl.ds(half, half)]],
                                  rowsb.at[pl.ds(half, half)], gsem).wait()

        def start_scatter(c, rowsb, ssem):
            t0 = tok0 + c * GCH
            pltpu.async_copy(rowsb, emb_hbm.at[pl.ds(t0, GCH)], ssem)

        def wait_scatter(c, rowsb, ssem):
            t0 = tok0 + c * GCH
            pltpu.make_async_copy(rowsb, emb_hbm.at[pl.ds(t0, GCH)],
                                  ssem).wait()

        def step(c, b):
            wait_gather(idxs[b], rows[b], gsems[b])
            start_scatter(c, rows[b], ssems[b])
            g = c + 2
            b2 = (b + 2) % 3
            if isinstance(g, int) and g >= nchunk:
                return

            def issue():
                start_gather(g, idxs[b2], rows[b2], gsems[b2])

            if isinstance(g, int):
                if g >= 3:
                    wait_scatter(g - 3, rows[b2], ssems[b2])
                issue()
            else:
                @pl.when(g >= 3)
                def _():
                    wait_scatter(g - 3, rows[b2], ssems[b2])
                issue()

        start_gather(0, idx0, rows0, gsem0)
        start_gather(1, idx1, rows1, gsem1)

        nsteady = (nchunk - 2) // 3
        def outer(k, carry):
            c0 = 3 * k
            step(c0, 0)
            step(c0 + 1, 1)
            step(c0 + 2, 2)
            return carry
        lax.fori_loop(0, nsteady, outer, 0)
        for c in range(nsteady * 3, nchunk):
            step(c, c % 3)
        for c in range(nchunk - 3, nchunk):
            wait_scatter(c, rows[c % 3], ssems[c % 3])

    return pl.kernel(
        body,
        out_type=jax.ShapeDtypeStruct((tokens, DIM), jnp.float32),
        mesh=mesh,
        scratch_types=[
            pltpu.VMEM((GCH,), jnp.int32),
            pltpu.VMEM((GCH,), jnp.int32),
            pltpu.VMEM((GCH,), jnp.int32),
            pltpu.VMEM((GCH, DIM), jnp.float32),
            pltpu.VMEM((GCH, DIM), jnp.float32),
            pltpu.VMEM((GCH, DIM), jnp.float32),
            pltpu.SemaphoreType.DMA,
            pltpu.SemaphoreType.DMA,
            pltpu.SemaphoreType.DMA,
            pltpu.SemaphoreType.DMA,
            pltpu.SemaphoreType.DMA,
            pltpu.SemaphoreType.DMA,
        ],
    )


def _ln_body(emb_ref, pe_ref, cst_ref, gam_ref, bet_ref, out_ref):
    inv = cst_ref[0, 0]
    zp = cst_ref[0, 1]
    scale = cst_ref[0, 2]
    x = emb_ref[...]
    q = jnp.round(x * inv + zp)
    e = q * scale + pe_ref[...]
    mean = jnp.mean(e, axis=-1, keepdims=True)
    var = jnp.mean(e * e, axis=-1, keepdims=True) - mean * mean
    r = lax.rsqrt(var + 1e-5)
    out_ref[...] = (e - mean) * r * gam_ref[...] + bet_ref[...]


def _ln_pass(emb, pe_blk, cst, gamma, beta, tokens, seq):
    blk = seq * SEQ_PER_BLK
    grid = tokens // blk
    assert tokens % blk == 0
    return pl.pallas_call(
        _ln_body,
        grid=(grid,),
        in_specs=[
            pl.BlockSpec((blk, DIM), lambda i: (i, 0)),
            pl.BlockSpec((blk, DIM), lambda i: (0, 0)),
            pl.BlockSpec((1, 3), lambda i: (0, 0), memory_space=pltpu.SMEM),
            pl.BlockSpec((1, DIM), lambda i: (0, 0)),
            pl.BlockSpec((1, DIM), lambda i: (0, 0)),
        ],
        out_specs=pl.BlockSpec((blk, DIM), lambda i: (i, 0)),
        out_shape=jax.ShapeDtypeStruct((tokens, DIM), jnp.float32),
    )(emb, pe_blk, cst, gamma, beta)


def kernel(input_ids, weight, gamma, beta):
    batch, seq = input_ids.shape
    tokens = batch * seq
    pe = _positional_table(seq)

    wmin, wmax = _weight_minmax(weight)
    scale = (wmax - wmin) / 255.0
    zp = -128.0 - wmin / scale
    cst = jnp.stack([1.0 / scale, zp, scale]).reshape(1, 3)
    pe_blk = jnp.tile(pe, (SEQ_PER_BLK, 1)) - zp * scale

    ids_flat = input_ids.reshape(tokens).astype(jnp.int32)
    gather = _make_sc_gather(tokens)
    emb = gather(ids_flat, weight)
    out = _ln_pass(emb, pe_blk, cst, gamma.reshape(1, DIM),
                   beta.reshape(1, DIM), tokens, seq)
    return out.reshape(batch, seq, DIM)


# speedup vs baseline: 9.9805x; 1.0250x over previous
import numpy as np
import jax
import jax.numpy as jnp
from jax import lax
from jax.experimental import pallas as pl
from jax.experimental.pallas import tpu as pltpu
from jax.experimental.pallas import tpu_sc as plsc

VOCAB = 100000
DIM = 128
BASE = 512
NC, NS = 2, 16
NW = NC * NS
LANES = 16
GCH = 256
SEQ_PER_BLK = 64


def _positional_table(seq_len):
    position = np.arange(BASE, dtype=np.float32)[:, None]
    div_term = np.exp(
        np.arange(0, DIM, 2, dtype=np.float32) * (-np.log(10000.0) / DIM))
    pe = np.zeros((BASE, DIM), dtype=np.float32)
    pe[:, 0::2] = np.sin(position * div_term)
    pe[:, 1::2] = np.cos(position * div_term)
    return jnp.asarray(pe[:seq_len])


def _minmax_body(w_ref, mn_ref, mx_ref):
    i = pl.program_id(0)
    bmn = jnp.min(w_ref[...])
    bmx = jnp.max(w_ref[...])

    @pl.when(i == 0)
    def _():
        mn_ref[0, 0] = bmn
        mx_ref[0, 0] = bmx

    @pl.when(i != 0)
    def _():
        mn_ref[0, 0] = jnp.minimum(mn_ref[0, 0], bmn)
        mx_ref[0, 0] = jnp.maximum(mx_ref[0, 0], bmx)


def _weight_minmax(weight):
    rows = 2000
    grid = VOCAB // rows
    mn, mx = pl.pallas_call(
        _minmax_body,
        grid=(grid,),
        in_specs=[pl.BlockSpec((rows, DIM), lambda i: (i, 0))],
        out_specs=[
            pl.BlockSpec((1, 1), lambda i: (0, 0), memory_space=pltpu.SMEM),
            pl.BlockSpec((1, 1), lambda i: (0, 0), memory_space=pltpu.SMEM),
        ],
        out_shape=[
            jax.ShapeDtypeStruct((1, 1), jnp.float32),
            jax.ShapeDtypeStruct((1, 1), jnp.float32),
        ],
    )(weight)
    return mn[0, 0], mx[0, 0]


def _make_sc_gather(tokens):
    tok_w = tokens // NW
    nchunk = tok_w // GCH
    assert tokens % NW == 0 and tok_w % GCH == 0 and nchunk >= 3
    half = GCH // 2

    mesh = plsc.VectorSubcoreMesh(core_axis_name="c", subcore_axis_name="s")

    def body(ids_hbm, w_hbm, emb_hbm,
             idv, rows0, rows1, rows2,
             gsem0, gsem1, gsem2, ssem0, ssem1, ssem2):
        rows = (rows0, rows1, rows2)
        gsems = (gsem0, gsem1, gsem2)
        ssems = (ssem0, ssem1, ssem2)

        cid = lax.axis_index("c")
        sid = lax.axis_index("s")
        wid = sid * NC + cid
        tok0 = wid * tok_w

        pltpu.sync_copy(ids_hbm.at[pl.ds(tok0, tok_w)], idv)

        def start_gather(c, rowsb, gsem):
            o = c * GCH
            pltpu.async_copy(w_hbm.at[idv.at[pl.ds(o, half)]],
                             rowsb.at[pl.ds(0, half)], gsem)
            pltpu.async_copy(w_hbm.at[idv.at[pl.ds(o + half, half)]],
                             rowsb.at[pl.ds(half, half)], gsem)

        def wait_gather(c, rowsb, gsem):
            o = c * GCH
            pltpu.make_async_copy(w_hbm.at[idv.at[pl.ds(o, half)]],
                                  rowsb.at[pl.ds(0, half)], gsem).wait()
            pltpu.make_async_copy(w_hbm.at[idv.at[pl.ds(o + half, half)]],
                                  rowsb.at[pl.ds(half, half)], gsem).wait()

        def start_scatter(c, rowsb, ssem):
            t0 = tok0 + c * GCH
            pltpu.async_copy(rowsb, emb_hbm.at[pl.ds(t0, GCH)], ssem)

        def wait_scatter(c, rowsb, ssem):
            t0 = tok0 + c * GCH
            pltpu.make_async_copy(rowsb, emb_hbm.at[pl.ds(t0, GCH)],
                                  ssem).wait()

        def step(c, b):
            wait_gather(c, rows[b], gsems[b])
            start_scatter(c, rows[b], ssems[b])
            g = c + 2
            b2 = (b + 2) % 3
            if isinstance(g, int) and g >= nchunk:
                return

            def issue():
                start_gather(g, rows[b2], gsems[b2])

            if isinstance(g, int):
                if g >= 3:
                    wait_scatter(g - 3, rows[b2], ssems[b2])
                issue()
            else:
                @pl.when(g >= 3)
                def _():
                    wait_scatter(g - 3, rows[b2], ssems[b2])
                issue()

        start_gather(0, rows0, gsem0)
        start_gather(1, rows1, gsem1)

        nsteady = (nchunk - 2) // 3
        def outer(k, carry):
            c0 = 3 * k
            step(c0, 0)
            step(c0 + 1, 1)
            step(c0 + 2, 2)
            return carry
        lax.fori_loop(0, nsteady, outer, 0)
        for c in range(nsteady * 3, nchunk):
            step(c, c % 3)
        for c in range(nchunk - 3, nchunk):
            wait_scatter(c, rows[c % 3], ssems[c % 3])

    return pl.kernel(
        body,
        out_type=jax.ShapeDtypeStruct((tokens, DIM), jnp.float32),
        mesh=mesh,
        scratch_types=[
            pltpu.VMEM((tok_w,), jnp.int32),
            pltpu.VMEM((GCH, DIM), jnp.float32),
            pltpu.VMEM((GCH, DIM), jnp.float32),
            pltpu.VMEM((GCH, DIM), jnp.float32),
            pltpu.SemaphoreType.DMA,
            pltpu.SemaphoreType.DMA,
            pltpu.SemaphoreType.DMA,
            pltpu.SemaphoreType.DMA,
            pltpu.SemaphoreType.DMA,
            pltpu.SemaphoreType.DMA,
        ],
    )


def _ln_body(emb_ref, pe_ref, cst_ref, gam_ref, bet_ref, out_ref):
    inv = cst_ref[0, 0]
    zp = cst_ref[0, 1]
    scale = cst_ref[0, 2]
    x = emb_ref[...]
    q = jnp.round(x * inv + zp)
    e = q * scale + pe_ref[...]
    mean = jnp.mean(e, axis=-1, keepdims=True)
    var = jnp.mean(e * e, axis=-1, keepdims=True) - mean * mean
    r = lax.rsqrt(var + 1e-5)
    out_ref[...] = (e - mean) * r * gam_ref[...] + bet_ref[...]


def _ln_pass(emb3, pe3, cst, gamma, beta, batch, seq):
    grid = batch // SEQ_PER_BLK
    assert batch % SEQ_PER_BLK == 0
    return pl.pallas_call(
        _ln_body,
        grid=(grid,),
        in_specs=[
            pl.BlockSpec((SEQ_PER_BLK, seq, DIM), lambda i: (i, 0, 0)),
            pl.BlockSpec((1, seq, DIM), lambda i: (0, 0, 0)),
            pl.BlockSpec((1, 3), lambda i: (0, 0), memory_space=pltpu.SMEM),
            pl.BlockSpec((1, 1, DIM), lambda i: (0, 0, 0)),
            pl.BlockSpec((1, 1, DIM), lambda i: (0, 0, 0)),
        ],
        out_specs=pl.BlockSpec((SEQ_PER_BLK, seq, DIM), lambda i: (i, 0, 0)),
        out_shape=jax.ShapeDtypeStruct((batch, seq, DIM), jnp.float32),
    )(emb3, pe3, cst, gamma, beta)


def kernel(input_ids, weight, gamma, beta):
    batch, seq = input_ids.shape
    tokens = batch * seq
    pe = _positional_table(seq)

    wmin, wmax = _weight_minmax(weight)
    scale = (wmax - wmin) / 255.0
    zp = -128.0 - wmin / scale
    cst = jnp.stack([1.0 / scale, zp, scale]).reshape(1, 3)
    pe3 = (pe - zp * scale).reshape(1, seq, DIM)

    ids_flat = input_ids.reshape(tokens).astype(jnp.int32)
    gather = _make_sc_gather(tokens)
    emb = gather(ids_flat, weight)
    return _ln_pass(emb.reshape(batch, seq, DIM), pe3, cst,
                    gamma.reshape(1, 1, DIM), beta.reshape(1, 1, DIM),
                    batch, seq)


# speedup vs baseline: 10.0094x; 1.0029x over previous
import numpy as np
import jax
import jax.numpy as jnp
from jax import lax
from jax.experimental import pallas as pl
from jax.experimental.pallas import tpu as pltpu
from jax.experimental.pallas import tpu_sc as plsc

VOCAB = 100000
DIM = 128
BASE = 512
NC, NS = 2, 16
NW = NC * NS
LANES = 16
GCH = 128
SEQ_PER_BLK = 64


def _positional_table(seq_len):
    position = np.arange(BASE, dtype=np.float32)[:, None]
    div_term = np.exp(
        np.arange(0, DIM, 2, dtype=np.float32) * (-np.log(10000.0) / DIM))
    pe = np.zeros((BASE, DIM), dtype=np.float32)
    pe[:, 0::2] = np.sin(position * div_term)
    pe[:, 1::2] = np.cos(position * div_term)
    return jnp.asarray(pe[:seq_len])


def _minmax_body(w_ref, mn_ref, mx_ref):
    i = pl.program_id(0)
    bmn = jnp.min(w_ref[...])
    bmx = jnp.max(w_ref[...])

    @pl.when(i == 0)
    def _():
        mn_ref[0, 0] = bmn
        mx_ref[0, 0] = bmx

    @pl.when(i != 0)
    def _():
        mn_ref[0, 0] = jnp.minimum(mn_ref[0, 0], bmn)
        mx_ref[0, 0] = jnp.maximum(mx_ref[0, 0], bmx)


def _weight_minmax(weight):
    rows = 2000
    grid = VOCAB // rows
    mn, mx = pl.pallas_call(
        _minmax_body,
        grid=(grid,),
        in_specs=[pl.BlockSpec((rows, DIM), lambda i: (i, 0))],
        out_specs=[
            pl.BlockSpec((1, 1), lambda i: (0, 0), memory_space=pltpu.SMEM),
            pl.BlockSpec((1, 1), lambda i: (0, 0), memory_space=pltpu.SMEM),
        ],
        out_shape=[
            jax.ShapeDtypeStruct((1, 1), jnp.float32),
            jax.ShapeDtypeStruct((1, 1), jnp.float32),
        ],
    )(weight)
    return mn[0, 0], mx[0, 0]


def _make_sc_gather(tokens):
    tok_w = tokens // NW
    nchunk = tok_w // GCH
    nbuf = 4
    assert tokens % NW == 0 and tok_w % GCH == 0 and nchunk >= nbuf

    mesh = plsc.VectorSubcoreMesh(core_axis_name="c", subcore_axis_name="s")

    def body(ids_hbm, w_hbm, emb_hbm,
             idv, rows0, rows1, rows2, rows3,
             gsem0, gsem1, gsem2, gsem3, ssem0, ssem1, ssem2, ssem3):
        rows = (rows0, rows1, rows2, rows3)
        gsems = (gsem0, gsem1, gsem2, gsem3)
        ssems = (ssem0, ssem1, ssem2, ssem3)

        cid = lax.axis_index("c")
        sid = lax.axis_index("s")
        wid = sid * NC + cid
        tok0 = wid * tok_w

        pltpu.sync_copy(ids_hbm.at[pl.ds(tok0, tok_w)], idv)

        def start_gather(c, rowsb, gsem):
            pltpu.async_copy(w_hbm.at[idv.at[pl.ds(c * GCH, GCH)]],
                             rowsb, gsem)

        def wait_gather(c, rowsb, gsem):
            pltpu.make_async_copy(w_hbm.at[idv.at[pl.ds(c * GCH, GCH)]],
                                  rowsb, gsem).wait()

        def start_scatter(c, rowsb, ssem):
            t0 = tok0 + c * GCH
            pltpu.async_copy(rowsb, emb_hbm.at[pl.ds(t0, GCH)], ssem)

        def wait_scatter(c, rowsb, ssem):
            t0 = tok0 + c * GCH
            pltpu.make_async_copy(rowsb, emb_hbm.at[pl.ds(t0, GCH)],
                                  ssem).wait()

        def step(c, b):
            wait_gather(c, rows[b], gsems[b])
            start_scatter(c, rows[b], ssems[b])
            g = c + nbuf - 1
            b2 = (b + nbuf - 1) % nbuf
            if isinstance(g, int) and g >= nchunk:
                return

            def issue():
                start_gather(g, rows[b2], gsems[b2])

            if isinstance(g, int):
                if g >= nbuf:
                    wait_scatter(g - nbuf, rows[b2], ssems[b2])
                issue()
            else:
                @pl.when(g >= nbuf)
                def _():
                    wait_scatter(g - nbuf, rows[b2], ssems[b2])
                issue()

        for c in range(nbuf - 1):
            start_gather(c, rows[c], gsems[c])

        nsteady = (nchunk - (nbuf - 1)) // nbuf
        def outer(k, carry):
            c0 = nbuf * k
            for b in range(nbuf):
                step(c0 + b, b)
            return carry
        lax.fori_loop(0, nsteady, outer, 0)
        for c in range(nsteady * nbuf, nchunk):
            step(c, c % nbuf)
        for c in range(nchunk - nbuf, nchunk):
            wait_scatter(c, rows[c % nbuf], ssems[c % nbuf])

    return pl.kernel(
        body,
        out_type=jax.ShapeDtypeStruct((tokens, DIM), jnp.float32),
        mesh=mesh,
        scratch_types=[
            pltpu.VMEM((tok_w,), jnp.int32),
            pltpu.VMEM((GCH, DIM), jnp.float32),
            pltpu.VMEM((GCH, DIM), jnp.float32),
            pltpu.VMEM((GCH, DIM), jnp.float32),
            pltpu.VMEM((GCH, DIM), jnp.float32),
            pltpu.SemaphoreType.DMA,
            pltpu.SemaphoreType.DMA,
            pltpu.SemaphoreType.DMA,
            pltpu.SemaphoreType.DMA,
            pltpu.SemaphoreType.DMA,
            pltpu.SemaphoreType.DMA,
            pltpu.SemaphoreType.DMA,
            pltpu.SemaphoreType.DMA,
        ],
    )


def _ln_body(emb_ref, pe_ref, cst_ref, gam_ref, bet_ref, out_ref):
    inv = cst_ref[0, 0]
    zp = cst_ref[0, 1]
    scale = cst_ref[0, 2]
    x = emb_ref[...]
    q = jnp.round(x * inv + zp)
    e = q * scale + pe_ref[...]
    mean = jnp.mean(e, axis=-1, keepdims=True)
    var = jnp.mean(e * e, axis=-1, keepdims=True) - mean * mean
    r = lax.rsqrt(var + 1e-5)
    out_ref[...] = (e - mean) * r * gam_ref[...] + bet_ref[...]


def _ln_pass(emb3, pe3, cst, gamma, beta, batch, seq):
    grid = batch // SEQ_PER_BLK
    assert batch % SEQ_PER_BLK == 0
    return pl.pallas_call(
        _ln_body,
        grid=(grid,),
        in_specs=[
            pl.BlockSpec((SEQ_PER_BLK, seq, DIM), lambda i: (i, 0, 0)),
            pl.BlockSpec((1, seq, DIM), lambda i: (0, 0, 0)),
            pl.BlockSpec((1, 3), lambda i: (0, 0), memory_space=pltpu.SMEM),
            pl.BlockSpec((1, 1, DIM), lambda i: (0, 0, 0)),
            pl.BlockSpec((1, 1, DIM), lambda i: (0, 0, 0)),
        ],
        out_specs=pl.BlockSpec((SEQ_PER_BLK, seq, DIM), lambda i: (i, 0, 0)),
        out_shape=jax.ShapeDtypeStruct((batch, seq, DIM), jnp.float32),
    )(emb3, pe3, cst, gamma, beta)


def kernel(input_ids, weight, gamma, beta):
    batch, seq = input_ids.shape
    tokens = batch * seq
    pe = _positional_table(seq)

    wmin, wmax = _weight_minmax(weight)
    scale = (wmax - wmin) / 255.0
    zp = -128.0 - wmin / scale
    cst = jnp.stack([1.0 / scale, zp, scale]).reshape(1, 3)
    pe3 = (pe - zp * scale).reshape(1, seq, DIM)

    ids_flat = input_ids.reshape(tokens).astype(jnp.int32)
    gather = _make_sc_gather(tokens)
    emb = gather(ids_flat, weight)
    return _ln_pass(emb.reshape(batch, seq, DIM), pe3, cst,
                    gamma.reshape(1, 1, DIM), beta.reshape(1, 1, DIM),
                    batch, seq)
